# baseline (speedup 1.0000x reference)
import functools

import jax
import jax.numpy as jnp
from jax import lax
from jax.experimental import pallas as pl
from jax.experimental.pallas import tpu as pltpu
from jax.experimental.pallas import tpu_sc as plsc

L = 16
NUM_CORES = 2
NUM_SUBCORES = 16
NW = NUM_CORES * NUM_SUBCORES

W = 1152
N_FULL = 868
TAIL_START = N_FULL * W
TAIL_N = 64
NTILES = W // 128
RING = 32


def _make_extract(n, v):
    mesh = plsc.VectorSubcoreMesh(core_axis_name="c", subcore_axis_name="s")
    n_vec = n // L

    @functools.partial(
        pl.kernel,
        mesh=mesh,
        compiler_params=pltpu.CompilerParams(needs_layout_passes=False),
        out_type=(
            jax.ShapeDtypeStruct((n * L,), jnp.float32),
            jax.ShapeDtypeStruct((n * L,), jnp.float32),
        ),
        scratch_types=[
            pltpu.VMEM((n,), jnp.int32),
            pltpu.VMEM((n,), jnp.int32),
            pltpu.VMEM((L, W), jnp.float32),
            pltpu.VMEM((L, W), jnp.float32),
            pltpu.VMEM((L, W), jnp.float32),
            pltpu.VMEM((L, W), jnp.float32),
            pltpu.VMEM((n,), jnp.int32),
            pltpu.VMEM((RING * L,), jnp.float32),
            pltpu.SemaphoreType.DMA,
            pltpu.SemaphoreType.DMA,
            pltpu.SemaphoreType.DMA,
            pltpu.SemaphoreType.DMA,
            pltpu.SemaphoreType.DMA,
        ],
    )
    def ka(users_hbm, items_hbm, utt_hbm, itt_hbm, ubuf_hbm, ibuf_hbm,
           uidx_v, iidx_v, ub0, ub1, ib0, ib1, hits_v, ring_v,
           us0, us1, is0, is1, wsem):
        wid = lax.axis_index("s") * NUM_CORES + lax.axis_index("c")
        pltpu.sync_copy(users_hbm, uidx_v)
        pltpu.sync_copy(items_hbm, iidx_v)

        ubufs, usems = (ub0, ub1), (us0, us1)
        ibufs, isems = (ib0, ib1), (is0, is1)

        n_j = jnp.where(wid < N_FULL % NW, N_FULL // NW + 1, N_FULL // NW)
        pos_iota = lax.iota(jnp.int32, L)
        c_iota = lax.iota(jnp.int32, L)
        lane2048 = pos_iota * 2048

        def bias(idx_ref):
            def bbody(jv, carry):
                sl = pl.ds(jv * L, L)
                idx_ref[sl] = idx_ref[sl] + (jv * (L * 2048) + lane2048)
                return carry
            lax.fori_loop(0, n_vec, bbody, jnp.int32(0))

        bias(uidx_v)
        bias(iidx_v)

        def fire2(cid, slot):
            col = cid * W
            pltpu.async_copy(utt_hbm.at[:, pl.ds(col, W)],
                             ubufs[slot], usems[slot])
            pltpu.async_copy(itt_hbm.at[:, pl.ds(col, W)],
                             ibufs[slot], isems[slot])

        def wait2(slot):
            pltpu.make_async_copy(
                utt_hbm.at[:, pl.ds(0, W)], ubufs[slot], usems[slot]).wait()
            pltpu.make_async_copy(
                itt_hbm.at[:, pl.ds(0, W)], ibufs[slot], isems[slot]).wait()

        def scan_chunk(idx_ref, lo, width):
            UNR = 8

            def body(j8, ptr):
                masks, packs, cnts = [], [], []
                for u in range(UNR):
                    jv = j8 * UNR + u
                    pidx = idx_ref[pl.ds(jv * L, L)]
                    packed = pidx - lo
                    rel = (packed - lane2048) - jv * (L * 2048)
                    mask = (rel >= 0) & (rel < width)
                    masks.append(mask)
                    packs.append(packed)
                    cnts.append(plsc.all_reduce_population_count(mask)[0])
                off = ptr
                for u in range(UNR):
                    plsc.store_compressed(
                        hits_v.at[pl.ds(off, L)], packs[u], mask=masks[u])
                    off = off + cnts[u]
                return off

            return lax.fori_loop(0, n_vec // UNR, body, jnp.int32(0))

        def extract(buf, out_hbm, nh):
            def per16(k, r):
                packed = hits_v[pl.ds(k * L, L)]
                relv = packed & 2047
                posv = packed >> 11
                validi = jnp.where((k * L + pos_iota) < nh,
                                   jnp.int32(1), jnp.int32(0))

                def lane(l, r2):
                    rel = jnp.minimum(relv[l], W - 1)
                    pos = posv[l]
                    ok = validi[l] > 0
                    row = plsc.load_gather(
                        buf, [c_iota, jnp.full((L,), 0, jnp.int32) + rel])
                    slot = r2 % RING

                    @pl.when(ok & (r2 >= RING))
                    def _():
                        pltpu.make_async_copy(
                            ring_v.at[pl.ds(0, L)],
                            out_hbm.at[pl.ds(0, L)], wsem).wait()

                    ring_v[pl.ds(slot * L, L)] = row

                    @pl.when(ok)
                    def _():
                        pltpu.async_copy(
                            ring_v.at[pl.ds(slot * L, L)],
                            out_hbm.at[pl.ds(pos * L, L)], wsem)
                    return r2 + validi[l]

                for l in range(L):
                    r = lane(l, r)
                return r

            nvec_h = (nh + L - 1) // L
            total = lax.fori_loop(0, nvec_h, per16, jnp.int32(0))

            def drain(_, c):
                pltpu.make_async_copy(
                    ring_v.at[pl.ds(0, L)],
                    out_hbm.at[pl.ds(0, L)], wsem).wait()
                return c

            lax.fori_loop(0, jnp.minimum(total, RING), drain, jnp.int32(0))

        def cid_of(j):
            return j * NW + wid

        fire2(cid_of(0), 0)

        def body_slot(j, slot):
            @pl.when(j + 1 < n_j)
            def _():
                fire2(cid_of(j + 1), 1 - slot)

            lo = cid_of(j) * W
            wait2(slot)
            nh_u = scan_chunk(uidx_v, lo, W)
            extract(ubufs[slot], ubuf_hbm, nh_u)
            nh_i = scan_chunk(iidx_v, lo, W)
            extract(ibufs[slot], ibuf_hbm, nh_i)

        def step(j, carry):
            @pl.when(j % 2 == 0)
            def _():
                body_slot(j, 0)

            @pl.when(j % 2 == 1)
            def _():
                body_slot(j, 1)

            return carry

        lax.fori_loop(0, n_j, step, jnp.int32(0))

    return ka


def _make_dot(n):
    mesh = plsc.VectorSubcoreMesh(core_axis_name="c", subcore_axis_name="s")
    n_per_w = n // NW

    @functools.partial(
        pl.kernel,
        mesh=mesh,
        compiler_params=pltpu.CompilerParams(
            needs_layout_passes=False, use_tc_tiling_on_sc=False),
        out_type=jax.ShapeDtypeStruct((n,), jnp.float32),
        scratch_types=[
            pltpu.VMEM((n_per_w * L,), jnp.float32),
            pltpu.VMEM((n_per_w * L,), jnp.float32),
            pltpu.VMEM((L,), jnp.float32),
            pltpu.VMEM((n_per_w,), jnp.float32),
            pltpu.VMEM((n_per_w,), jnp.int32),
            pltpu.VMEM((n_per_w,), jnp.int32),
            pltpu.VMEM((TAIL_N, L), jnp.float32),
            pltpu.VMEM((TAIL_N, L), jnp.float32),
        ],
    )
    def kb(ubuf_hbm, ibuf_hbm, users_hbm, items_hbm, tailu_hbm, taili_hbm,
           h_hbm, out_hbm, urows_v, irows_v, h_v, out_v, uix_v, iix_v,
           tailu_v, taili_v):
        wid = lax.axis_index("s") * NUM_CORES + lax.axis_index("c")
        base = wid * n_per_w
        pltpu.sync_copy(ubuf_hbm.at[pl.ds(base * L, n_per_w * L)], urows_v)
        pltpu.sync_copy(ibuf_hbm.at[pl.ds(base * L, n_per_w * L)], irows_v)
        pltpu.sync_copy(users_hbm.at[pl.ds(base, n_per_w)], uix_v)
        pltpu.sync_copy(items_hbm.at[pl.ds(base, n_per_w)], iix_v)
        pltpu.sync_copy(tailu_hbm, tailu_v)
        pltpu.sync_copy(taili_hbm, taili_v)
        pltpu.sync_copy(h_hbm, h_v)
        pos16 = lax.iota(jnp.int32, L)

        def patch_tail(ix_ref, tail_v, rows_v):
            def group(g, carry):
                idx = ix_ref[pl.ds(g * L, L)]
                tmask = idx >= TAIL_START
                rel = jnp.clip(idx - TAIL_START, 0, TAIL_N - 1)
                flat = (g * L + pos16) * L
                for c in range(L):
                    vals = plsc.load_gather(
                        tail_v, [rel, jnp.full((L,), c, jnp.int32)])
                    plsc.store_scatter(
                        rows_v, [flat + c], vals, mask=tmask)
                return carry

            lax.fori_loop(0, n_per_w // L, group, jnp.int32(0))

        patch_tail(uix_v, tailu_v, urows_v)
        patch_tail(iix_v, taili_v, irows_v)
        hv = h_v[...]
        hs = [hv[c] for c in range(L)]
        row_iota = lax.iota(jnp.int32, L)

        def body(g, carry):
            flat0 = (g * L + row_iota) * L
            acc = jnp.zeros((L,), jnp.float32)
            for c in range(L):
                u = plsc.load_gather(urows_v, [flat0 + c])
                i = plsc.load_gather(irows_v, [flat0 + c])
                acc = acc + u * i * hs[c]
            out_v[pl.ds(g * L, L)] = acc
            return carry

        lax.fori_loop(0, n_per_w // L, body, 0)
        pltpu.sync_copy(out_v, out_hbm.at[pl.ds(base, n_per_w)])

    return kb


def kernel(users, items, user_table, item_table, h):
    n = users.shape[0]
    v = user_table.shape[0]
    utt, itt = user_table.T, item_table.T
    ubuf, ibuf = _make_extract(n, v)(users, items, utt, itt)
    return _make_dot(n)(ubuf, ibuf, users, items,
                        user_table[TAIL_START:], item_table[TAIL_START:], h)


# speedup vs baseline: 2.0189x; 2.0189x over previous
import functools

import jax
import jax.numpy as jnp
from jax import lax
from jax.experimental import pallas as pl
from jax.experimental.pallas import tpu as pltpu
from jax.experimental.pallas import tpu_sc as plsc

L = 16
NUM_CORES = 2
NUM_SUBCORES = 16
NW = NUM_CORES * NUM_SUBCORES

W = 1152
N_FULL = 868
TAIL_START = N_FULL * W
TAIL_N = 64
NTILES = W // 128
RING = 32


def _make_extract(n, v):
    mesh = plsc.VectorSubcoreMesh(core_axis_name="c", subcore_axis_name="s")
    n_vec = n // L

    @functools.partial(
        pl.kernel,
        mesh=mesh,
        compiler_params=pltpu.CompilerParams(needs_layout_passes=False),
        out_type=(
            jax.ShapeDtypeStruct((n * L,), jnp.float32),
            jax.ShapeDtypeStruct((n * L,), jnp.float32),
        ),
        scratch_types=[
            pltpu.VMEM((n,), jnp.int32),
            pltpu.VMEM((n,), jnp.int32),
            pltpu.VMEM((n,), jnp.int32),
            pltpu.VMEM((L, W), jnp.float32),
            pltpu.VMEM((L, W), jnp.float32),
            pltpu.VMEM((L, W), jnp.float32),
            pltpu.VMEM((L, W), jnp.float32),
            pltpu.VMEM((RING * L,), jnp.float32),
            pltpu.SemaphoreType.DMA,
            pltpu.SemaphoreType.DMA,
            pltpu.SemaphoreType.DMA,
            pltpu.SemaphoreType.DMA,
            pltpu.SemaphoreType.DMA,
        ],
    )
    def ka(users_hbm, items_hbm, utt_hbm, itt_hbm, ubuf_hbm, ibuf_hbm,
           hits_v, uflt_v, iflt_v, ub0, ub1, ib0, ib1, ring_v,
           us0, us1, is0, is1, wsem):
        wid = lax.axis_index("s") * NUM_CORES + lax.axis_index("c")

        ubufs, usems = (ub0, ub1), (us0, us1)
        ibufs, isems = (ib0, ib1), (is0, is1)

        base_cnt = N_FULL // NW
        extra = jnp.minimum(wid, N_FULL % NW)
        start_chunk = wid * base_cnt + extra
        n_j = base_cnt + jnp.where(wid < N_FULL % NW, 1, 0)
        w_lo = start_chunk * W
        range_w = n_j * W
        pos_iota = lax.iota(jnp.int32, L)
        c_iota = lax.iota(jnp.int32, L)

        def prefilter(flt_ref):
            UNR = 8

            def body(j8, ptr):
                masks, packs, cnts = [], [], []
                for u in range(UNR):
                    jv = j8 * UNR + u
                    idx = hits_v[pl.ds(jv * L, L)]
                    rel_w = idx - w_lo
                    mask = (rel_w >= 0) & (rel_w < range_w)
                    masks.append(mask)
                    packs.append(rel_w + (jv * L + pos_iota) * 32768)
                    cnts.append(plsc.all_reduce_population_count(mask)[0])
                off = ptr
                for u in range(UNR):
                    plsc.store_compressed(
                        flt_ref.at[pl.ds(off, L)], packs[u], mask=masks[u])
                    off = off + cnts[u]
                return off

            return lax.fori_loop(0, n_vec // UNR, body, jnp.int32(0))

        pltpu.sync_copy(users_hbm, hits_v)
        nu_f = prefilter(uflt_v)
        pltpu.sync_copy(items_hbm, hits_v)
        ni_f = prefilter(iflt_v)

        def fire2(cid, slot):
            col = cid * W
            pltpu.async_copy(utt_hbm.at[:, pl.ds(col, W)],
                             ubufs[slot], usems[slot])
            pltpu.async_copy(itt_hbm.at[:, pl.ds(col, W)],
                             ibufs[slot], isems[slot])

        def wait2(slot):
            pltpu.make_async_copy(
                utt_hbm.at[:, pl.ds(0, W)], ubufs[slot], usems[slot]).wait()
            pltpu.make_async_copy(
                itt_hbm.at[:, pl.ds(0, W)], ibufs[slot], isems[slot]).wait()

        def scan_chunk(flt_ref, nf, lo_rel):
            def body(kv, ptr):
                pw = flt_ref[pl.ds(kv * L, L)]
                rel_w = pw & 32767
                pos = pw >> 15
                rel = rel_w - lo_rel
                mask = ((rel >= 0) & (rel < W)
                        & ((kv * L + pos_iota) < nf))
                packed = rel + pos * 2048
                plsc.store_compressed(
                    hits_v.at[pl.ds(ptr, L)], packed, mask=mask)
                cnt = plsc.all_reduce_population_count(mask)
                return ptr + cnt[0]

            return lax.fori_loop(0, (nf + L - 1) // L, body, jnp.int32(0))

        def extract(buf, out_hbm, nh):
            def per16(k, r):
                packed = hits_v[pl.ds(k * L, L)]
                relv = packed & 2047
                posv = packed >> 11
                validi = jnp.where((k * L + pos_iota) < nh,
                                   jnp.int32(1), jnp.int32(0))

                def lane(l, r2):
                    rel = jnp.minimum(relv[l], W - 1)
                    pos = posv[l]
                    ok = validi[l] > 0
                    row = plsc.load_gather(
                        buf, [c_iota, jnp.full((L,), 0, jnp.int32) + rel])
                    slot = r2 % RING

                    @pl.when(ok & (r2 >= RING))
                    def _():
                        pltpu.make_async_copy(
                            ring_v.at[pl.ds(0, L)],
                            out_hbm.at[pl.ds(0, L)], wsem).wait()

                    ring_v[pl.ds(slot * L, L)] = row

                    @pl.when(ok)
                    def _():
                        pltpu.async_copy(
                            ring_v.at[pl.ds(slot * L, L)],
                            out_hbm.at[pl.ds(pos * L, L)], wsem)
                    return r2 + validi[l]

                for l in range(L):
                    r = lane(l, r)
                return r

            nvec_h = (nh + L - 1) // L
            total = lax.fori_loop(0, nvec_h, per16, jnp.int32(0))

            def drain(_, c):
                pltpu.make_async_copy(
                    ring_v.at[pl.ds(0, L)],
                    out_hbm.at[pl.ds(0, L)], wsem).wait()
                return c

            lax.fori_loop(0, jnp.minimum(total, RING), drain, jnp.int32(0))

        def cid_of(j):
            return start_chunk + j

        fire2(cid_of(0), 0)

        def body_slot(j, slot):
            @pl.when(j + 1 < n_j)
            def _():
                fire2(cid_of(j + 1), 1 - slot)

            wait2(slot)
            nh_u = scan_chunk(uflt_v, nu_f, j * W)
            extract(ubufs[slot], ubuf_hbm, nh_u)
            nh_i = scan_chunk(iflt_v, ni_f, j * W)
            extract(ibufs[slot], ibuf_hbm, nh_i)

        def step(j, carry):
            @pl.when(j % 2 == 0)
            def _():
                body_slot(j, 0)

            @pl.when(j % 2 == 1)
            def _():
                body_slot(j, 1)

            return carry

        lax.fori_loop(0, n_j, step, jnp.int32(0))

    return ka


def _make_dot(n):
    mesh = plsc.VectorSubcoreMesh(core_axis_name="c", subcore_axis_name="s")
    n_per_w = n // NW

    @functools.partial(
        pl.kernel,
        mesh=mesh,
        compiler_params=pltpu.CompilerParams(
            needs_layout_passes=False, use_tc_tiling_on_sc=False),
        out_type=jax.ShapeDtypeStruct((n,), jnp.float32),
        scratch_types=[
            pltpu.VMEM((n_per_w * L,), jnp.float32),
            pltpu.VMEM((n_per_w * L,), jnp.float32),
            pltpu.VMEM((L,), jnp.float32),
            pltpu.VMEM((n_per_w,), jnp.float32),
            pltpu.VMEM((n_per_w,), jnp.int32),
            pltpu.VMEM((n_per_w,), jnp.int32),
            pltpu.VMEM((TAIL_N, L), jnp.float32),
            pltpu.VMEM((TAIL_N, L), jnp.float32),
        ],
    )
    def kb(ubuf_hbm, ibuf_hbm, users_hbm, items_hbm, tailu_hbm, taili_hbm,
           h_hbm, out_hbm, urows_v, irows_v, h_v, out_v, uix_v, iix_v,
           tailu_v, taili_v):
        wid = lax.axis_index("s") * NUM_CORES + lax.axis_index("c")
        base = wid * n_per_w
        pltpu.sync_copy(ubuf_hbm.at[pl.ds(base * L, n_per_w * L)], urows_v)
        pltpu.sync_copy(ibuf_hbm.at[pl.ds(base * L, n_per_w * L)], irows_v)
        pltpu.sync_copy(users_hbm.at[pl.ds(base, n_per_w)], uix_v)
        pltpu.sync_copy(items_hbm.at[pl.ds(base, n_per_w)], iix_v)
        pltpu.sync_copy(tailu_hbm, tailu_v)
        pltpu.sync_copy(taili_hbm, taili_v)
        pltpu.sync_copy(h_hbm, h_v)
        pos16 = lax.iota(jnp.int32, L)

        def patch_tail(ix_ref, tail_v, rows_v):
            def group(g, carry):
                idx = ix_ref[pl.ds(g * L, L)]
                tmask = idx >= TAIL_START
                rel = jnp.clip(idx - TAIL_START, 0, TAIL_N - 1)
                flat = (g * L + pos16) * L
                for c in range(L):
                    vals = plsc.load_gather(
                        tail_v, [rel, jnp.full((L,), c, jnp.int32)])
                    plsc.store_scatter(
                        rows_v, [flat + c], vals, mask=tmask)
                return carry

            lax.fori_loop(0, n_per_w // L, group, jnp.int32(0))

        patch_tail(uix_v, tailu_v, urows_v)
        patch_tail(iix_v, taili_v, irows_v)
        hv = h_v[...]
        hs = [hv[c] for c in range(L)]
        row_iota = lax.iota(jnp.int32, L)

        def body(g, carry):
            flat0 = (g * L + row_iota) * L
            acc = jnp.zeros((L,), jnp.float32)
            for c in range(L):
                u = plsc.load_gather(urows_v, [flat0 + c])
                i = plsc.load_gather(irows_v, [flat0 + c])
                acc = acc + u * i * hs[c]
            out_v[pl.ds(g * L, L)] = acc
            return carry

        lax.fori_loop(0, n_per_w // L, body, 0)
        pltpu.sync_copy(out_v, out_hbm.at[pl.ds(base, n_per_w)])

    return kb


def kernel(users, items, user_table, item_table, h):
    n = users.shape[0]
    v = user_table.shape[0]
    utt, itt = user_table.T, item_table.T
    ubuf, ibuf = _make_extract(n, v)(users, items, utt, itt)
    return _make_dot(n)(ubuf, ibuf, users, items,
                        user_table[TAIL_START:], item_table[TAIL_START:], h)
